# speedup vs baseline: 3.3366x; 3.3366x over previous
import functools

import jax
import jax.numpy as jnp
from jax import lax
from jax.experimental import pallas as pl
from jax.experimental.pallas import tpu as pltpu
from jax.experimental.pallas import tpu_sc as plsc

_NUM_CORES = 2
_NUM_SUBCORES = 16
_NW = _NUM_CORES * _NUM_SUBCORES
_CHUNK = 128


@jax.jit
def _embed(x3d, table):
    nw, n_chunks, chunk = x3d.shape
    vocab, d = table.shape
    n = nw * n_chunks * chunk
    per_w = n_chunks * chunk

    mesh = plsc.VectorSubcoreMesh(
        core_axis_name="c",
        subcore_axis_name="s",
        num_cores=_NUM_CORES,
        num_subcores=_NUM_SUBCORES,
    )

    @functools.partial(
        pl.kernel,
        out_type=jax.ShapeDtypeStruct((n, d), jnp.float32),
        mesh=mesh,
        scratch_types=[
            pltpu.VMEM((n_chunks, chunk), jnp.int32),
            pltpu.VMEM((chunk, d), jnp.float32),
            pltpu.VMEM((chunk, d), jnp.float32),
            pltpu.SemaphoreType.DMA,
            pltpu.SemaphoreType.DMA,
        ],
    )
    def embed_kernel(x_hbm, table_hbm, out_hbm, idx_v, buf0, buf1, sem0, sem1):
        wid = lax.axis_index("s") * _NUM_CORES + lax.axis_index("c")
        base = wid * per_w

        pltpu.sync_copy(x_hbm.at[wid], idx_v)

        pltpu.async_copy(table_hbm.at[idx_v.at[0]], buf0, sem0)
        pltpu.async_copy(table_hbm.at[idx_v.at[1]], buf1, sem1)

        bufs = (buf0, buf1)
        sems = (sem0, sem1)

        def outer(g, carry):
            for b in range(2):
                j = g * 2 + b
                buf = bufs[b]
                sem = sems[b]
                pltpu.make_async_copy(table_hbm.at[idx_v.at[j]], buf, sem).wait()
                pltpu.sync_copy(buf, out_hbm.at[pl.ds(base + j * chunk, chunk)])

                @pl.when(j + 2 < n_chunks)
                def _issue_next():
                    pltpu.async_copy(table_hbm.at[idx_v.at[j + 2]], buf, sem)

            return carry

        lax.fori_loop(0, n_chunks // 2, outer, None)

    return embed_kernel(x3d, table)


def kernel(x, table):
    batch, hist = x.shape
    n = batch * hist
    per_w = n // _NW
    n_chunks = per_w // _CHUNK
    x3d = x.reshape(_NW, n_chunks, _CHUNK).astype(jnp.int32)
    out = _embed(x3d, table)
    return out.reshape(batch, hist, table.shape[1])


# speedup vs baseline: 3.3397x; 1.0009x over previous
import functools

import jax
import jax.numpy as jnp
from jax import lax
from jax.experimental import pallas as pl
from jax.experimental.pallas import tpu as pltpu
from jax.experimental.pallas import tpu_sc as plsc

_NUM_CORES = 2
_NUM_SUBCORES = 16
_NW = _NUM_CORES * _NUM_SUBCORES
_CHUNK = 128


@jax.jit
def _embed(x3d, table):
    nw, n_chunks, chunk = x3d.shape
    vocab, d = table.shape
    n = nw * n_chunks * chunk
    per_w = n_chunks * chunk

    mesh = plsc.VectorSubcoreMesh(
        core_axis_name="c",
        subcore_axis_name="s",
        num_cores=_NUM_CORES,
        num_subcores=_NUM_SUBCORES,
    )

    nbuf = 5
    lookahead = 3

    @functools.partial(
        pl.kernel,
        out_type=jax.ShapeDtypeStruct((n, d), jnp.float32),
        mesh=mesh,
        scratch_types=[
            pltpu.VMEM((n_chunks, chunk), jnp.int32),
            [pltpu.VMEM((chunk, d), jnp.float32) for _ in range(nbuf)],
            [pltpu.SemaphoreType.DMA for _ in range(nbuf)],
            [pltpu.SemaphoreType.DMA for _ in range(nbuf)],
        ],
    )
    def embed_kernel(x_hbm, table_hbm, out_hbm, idx_v, bufs, sems_g, sems_s):
        wid = lax.axis_index("s") * _NUM_CORES + lax.axis_index("c")
        base = wid * per_w

        pltpu.sync_copy(x_hbm.at[wid], idx_v)

        for b in range(lookahead):
            pltpu.async_copy(table_hbm.at[idx_v.at[b]], bufs[b], sems_g[b])

        def outer(g, carry):
            for b in range(nbuf):
                j = g * nbuf + b
                pltpu.make_async_copy(
                    table_hbm.at[idx_v.at[j]], bufs[b], sems_g[b]
                ).wait()
                pltpu.async_copy(
                    bufs[b], out_hbm.at[pl.ds(base + j * chunk, chunk)], sems_s[b]
                )

                bf = (b + lookahead) % nbuf

                @pl.when(j + lookahead < n_chunks)
                def _prefetch():
                    @pl.when(j + lookahead >= nbuf)
                    def _drain_prev_scatter():
                        pltpu.make_async_copy(
                            bufs[bf],
                            out_hbm.at[pl.ds(base, chunk)],
                            sems_s[bf],
                        ).wait()

                    pltpu.async_copy(
                        table_hbm.at[idx_v.at[j + lookahead]], bufs[bf], sems_g[bf]
                    )

            return carry

        lax.fori_loop(0, n_chunks // nbuf, outer, None)

        for b in range(nbuf):
            pltpu.make_async_copy(
                bufs[b], out_hbm.at[pl.ds(base, chunk)], sems_s[b]
            ).wait()

    return embed_kernel(x3d, table)


def kernel(x, table):
    batch, hist = x.shape
    n = batch * hist
    per_w = n // _NW
    n_chunks = per_w // _CHUNK
    x3d = x.reshape(_NW, n_chunks, _CHUNK).astype(jnp.int32)
    out = _embed(x3d, table)
    return out.reshape(batch, hist, table.shape[1])


# speedup vs baseline: 3.3439x; 1.0013x over previous
import functools

import jax
import jax.numpy as jnp
from jax import lax
from jax.experimental import pallas as pl
from jax.experimental.pallas import tpu as pltpu
from jax.experimental.pallas import tpu_sc as plsc

_NUM_CORES = 2
_NUM_SUBCORES = 16
_NW = _NUM_CORES * _NUM_SUBCORES
_CHUNK = 128


@jax.jit
def _embed(x3d, table):
    nw, n_chunks, chunk = x3d.shape
    vocab, d = table.shape
    n = nw * n_chunks * chunk
    per_w = n_chunks * chunk

    mesh = plsc.VectorSubcoreMesh(
        core_axis_name="c",
        subcore_axis_name="s",
        num_cores=_NUM_CORES,
        num_subcores=_NUM_SUBCORES,
    )

    nbuf = 5
    lookahead = 3

    @functools.partial(
        pl.kernel,
        out_type=jax.ShapeDtypeStruct((n, d), jnp.float32),
        mesh=mesh,
        compiler_params=pltpu.CompilerParams(use_tc_tiling_on_sc=True),
        scratch_types=[
            pltpu.VMEM((n_chunks, chunk), jnp.int32),
            [pltpu.VMEM((chunk, d), jnp.float32) for _ in range(nbuf)],
            [pltpu.SemaphoreType.DMA for _ in range(nbuf)],
            [pltpu.SemaphoreType.DMA for _ in range(nbuf)],
        ],
    )
    def embed_kernel(x_hbm, table_hbm, out_hbm, idx_v, bufs, sems_g, sems_s):
        wid = lax.axis_index("s") * _NUM_CORES + lax.axis_index("c")
        base = wid * per_w

        pltpu.sync_copy(x_hbm.at[wid], idx_v)

        for b in range(lookahead):
            pltpu.async_copy(table_hbm.at[idx_v.at[b]], bufs[b], sems_g[b])

        def outer(g, carry):
            for b in range(nbuf):
                j = g * nbuf + b
                pltpu.make_async_copy(
                    table_hbm.at[idx_v.at[j]], bufs[b], sems_g[b]
                ).wait()
                pltpu.async_copy(
                    bufs[b], out_hbm.at[pl.ds(base + j * chunk, chunk)], sems_s[b]
                )

                bf = (b + lookahead) % nbuf

                @pl.when(j + lookahead < n_chunks)
                def _prefetch():
                    @pl.when(j + lookahead >= nbuf)
                    def _drain_prev_scatter():
                        pltpu.make_async_copy(
                            bufs[bf],
                            out_hbm.at[pl.ds(base, chunk)],
                            sems_s[bf],
                        ).wait()

                    pltpu.async_copy(
                        table_hbm.at[idx_v.at[j + lookahead]], bufs[bf], sems_g[bf]
                    )

            return carry

        lax.fori_loop(0, n_chunks // nbuf, outer, None)

        for b in range(nbuf):
            pltpu.make_async_copy(
                bufs[b], out_hbm.at[pl.ds(base, chunk)], sems_s[b]
            ).wait()

    return embed_kernel(x3d, table)


def kernel(x, table):
    batch, hist = x.shape
    n = batch * hist
    per_w = n // _NW
    n_chunks = per_w // _CHUNK
    x3d = x.reshape(_NW, n_chunks, _CHUNK).astype(jnp.int32)
    out = _embed(x3d, table)
    return out.reshape(batch, hist, table.shape[1])


# speedup vs baseline: 5.9674x; 1.7846x over previous
import functools

import jax
import jax.numpy as jnp
from jax import lax
from jax.experimental import pallas as pl
from jax.experimental.pallas import tpu as pltpu
from jax.experimental.pallas import tpu_sc as plsc

_NUM_CORES = 2
_NUM_SUBCORES = 16
_NW = _NUM_CORES * _NUM_SUBCORES


@jax.jit
def _embed(x3d, table):
    nw, b_per_w, hist = x3d.shape
    vocab, d = table.shape
    batch = nw * b_per_w

    mesh = plsc.VectorSubcoreMesh(
        core_axis_name="c",
        subcore_axis_name="s",
        num_cores=_NUM_CORES,
        num_subcores=_NUM_SUBCORES,
    )

    nbuf = 8
    lookahead = 4

    @functools.partial(
        pl.kernel,
        out_type=jax.ShapeDtypeStruct((batch, hist, d), jnp.float32),
        mesh=mesh,
        compiler_params=pltpu.CompilerParams(use_tc_tiling_on_sc=True),
        scratch_types=[
            pltpu.VMEM((b_per_w, hist), jnp.int32),
            [pltpu.VMEM((hist, d), jnp.float32) for _ in range(nbuf)],
            [pltpu.SemaphoreType.DMA for _ in range(nbuf)],
            [pltpu.SemaphoreType.DMA for _ in range(nbuf)],
        ],
    )
    def embed_kernel(x_hbm, table_hbm, out_hbm, idx_v, bufs, sems_g, sems_s):
        wid = lax.axis_index("s") * _NUM_CORES + lax.axis_index("c")
        base = wid * b_per_w

        pltpu.sync_copy(x_hbm.at[wid], idx_v)

        for b in range(lookahead):
            pltpu.async_copy(table_hbm.at[idx_v.at[b]], bufs[b], sems_g[b])

        def outer(g, carry):
            for b in range(nbuf):
                k = g * nbuf + b
                pltpu.make_async_copy(
                    table_hbm.at[idx_v.at[k]], bufs[b], sems_g[b]
                ).wait()
                pltpu.async_copy(bufs[b], out_hbm.at[base + k], sems_s[b])

                bf = (b + lookahead) % nbuf

                @pl.when(k + lookahead < b_per_w)
                def _prefetch():
                    @pl.when(k + lookahead >= nbuf)
                    def _drain_prev_scatter():
                        pltpu.make_async_copy(
                            bufs[bf], out_hbm.at[base], sems_s[bf]
                        ).wait()

                    pltpu.async_copy(
                        table_hbm.at[idx_v.at[k + lookahead]], bufs[bf], sems_g[bf]
                    )

            return carry

        lax.fori_loop(0, b_per_w // nbuf, outer, None)

        for b in range(nbuf):
            pltpu.make_async_copy(bufs[b], out_hbm.at[base], sems_s[b]).wait()

    return embed_kernel(x3d, table)


def kernel(x, table):
    batch, hist = x.shape
    x3d = x.reshape(_NW, batch // _NW, hist).astype(jnp.int32)
    return _embed(x3d, table)


# speedup vs baseline: 10.6631x; 1.7869x over previous
import functools

import jax
import jax.numpy as jnp
from jax import lax
from jax.experimental import pallas as pl
from jax.experimental.pallas import tpu as pltpu
from jax.experimental.pallas import tpu_sc as plsc

_NUM_CORES = 2
_NUM_SUBCORES = 16
_NW = _NUM_CORES * _NUM_SUBCORES


@jax.jit
def _embed(xw, table):
    nw, hist, b_per_w = xw.shape
    vocab, d = table.shape
    batch = nw * b_per_w

    mesh = plsc.VectorSubcoreMesh(
        core_axis_name="c",
        subcore_axis_name="s",
        num_cores=_NUM_CORES,
        num_subcores=_NUM_SUBCORES,
    )

    nbuf = 5
    lookahead = 3

    @functools.partial(
        pl.kernel,
        out_type=jax.ShapeDtypeStruct((hist * batch, d), jnp.float32),
        mesh=mesh,
        compiler_params=pltpu.CompilerParams(use_tc_tiling_on_sc=True),
        scratch_types=[
            pltpu.VMEM((hist, b_per_w), jnp.int32),
            [pltpu.VMEM((b_per_w, d), jnp.float32) for _ in range(nbuf)],
            [pltpu.SemaphoreType.DMA for _ in range(nbuf)],
            [pltpu.SemaphoreType.DMA for _ in range(nbuf)],
        ],
    )
    def embed_kernel(x_hbm, table_hbm, out_hbm, idx_v, bufs, sems_g, sems_s):
        wid = lax.axis_index("s") * _NUM_CORES + lax.axis_index("c")
        base = wid * b_per_w

        pltpu.sync_copy(x_hbm.at[wid], idx_v)

        for b in range(lookahead):
            pltpu.async_copy(table_hbm.at[idx_v.at[b]], bufs[b], sems_g[b])

        def outer(g, carry):
            for b in range(nbuf):
                h = g * nbuf + b
                pltpu.make_async_copy(
                    table_hbm.at[idx_v.at[h]], bufs[b], sems_g[b]
                ).wait()
                pltpu.async_copy(
                    bufs[b], out_hbm.at[pl.ds(h * batch + base, b_per_w)], sems_s[b]
                )

                bf = (b + lookahead) % nbuf

                @pl.when(h + lookahead < hist)
                def _prefetch():
                    @pl.when(h + lookahead >= nbuf)
                    def _drain_prev_scatter():
                        pltpu.make_async_copy(
                            bufs[bf],
                            out_hbm.at[pl.ds(base, b_per_w)],
                            sems_s[bf],
                        ).wait()

                    pltpu.async_copy(
                        table_hbm.at[idx_v.at[h + lookahead]], bufs[bf], sems_g[bf]
                    )

            return carry

        lax.fori_loop(0, hist // nbuf, outer, None)

        for b in range(nbuf):
            pltpu.make_async_copy(
                bufs[b], out_hbm.at[pl.ds(base, b_per_w)], sems_s[b]
            ).wait()

    return embed_kernel(xw, table)


def kernel(x, table):
    batch, hist = x.shape
    xw = x.astype(jnp.int32).T.reshape(hist, _NW, batch // _NW).transpose(1, 0, 2)
    out = _embed(xw, table)
    return out.reshape(hist, batch, table.shape[1]).swapaxes(0, 1)
